# baseline (speedup 1.0000x reference)
import jax
import jax.numpy as jnp
from jax.experimental import pallas as pl

B, P, N, C, H = 16, 512, 20, 9, 256
SENT = -1073741824.0


def _mlp_pool_kernel(x_ref, m_ref, w1_ref, b1_ref, w2_ref, b2_ref, o_ref):
    x = x_ref[0]
    ms = jnp.where(m_ref[0], 0.0, SENT)
    w1 = w1_ref[...].astype(jnp.bfloat16)
    w2 = w2_ref[...].astype(jnp.bfloat16)
    b1 = b1_ref[...].astype(jnp.bfloat16)
    acc = None
    for j in range(N):
        xj = x[:, j * C : (j + 1) * C].astype(jnp.bfloat16)
        d1 = jnp.dot(xj, w1, preferred_element_type=jnp.float32)
        h1 = jnp.maximum(d1.astype(jnp.bfloat16) + b1, jnp.bfloat16(0.0))
        g2 = jnp.dot(h1, w2, preferred_element_type=jnp.float32)
        cand = g2 + ms[:, j : j + 1]
        acc = cand if acc is None else jnp.maximum(acc, cand)
    o_ref[...] = jnp.where(acc < SENT / 2, 0.0, acc + b2_ref[...]).reshape(1, P, H)


@jax.jit
def kernel(polylines, polylines_mask, W1, b1, W2, b2):
    x = polylines.reshape(B, P, N * C)
    b1r = b1.reshape(1, H)
    b2r = b2.reshape(1, H)
    out = pl.pallas_call(
        _mlp_pool_kernel,
        grid=(B,),
        in_specs=[
            pl.BlockSpec((1, P, N * C), lambda g: (g, 0, 0)),
            pl.BlockSpec((1, P, N), lambda g: (g, 0, 0)),
            pl.BlockSpec((C, H), lambda g: (0, 0)),
            pl.BlockSpec((1, H), lambda g: (0, 0)),
            pl.BlockSpec((H, H), lambda g: (0, 0)),
            pl.BlockSpec((1, H), lambda g: (0, 0)),
        ],
        out_specs=pl.BlockSpec((1, P, H), lambda g: (g, 0, 0)),
        out_shape=jax.ShapeDtypeStruct((B, P, H), jnp.float32),
    )(x, polylines_mask, W1, b1r, W2, b2r)
    return out


# speedup vs baseline: 1.0331x; 1.0331x over previous
import jax
import jax.numpy as jnp
from jax.experimental import pallas as pl

B, P, N, C, H = 16, 512, 20, 9, 256
SENT = -1073741824.0
BB = 2


def _mlp_pool_kernel(x_ref, m_ref, w1_ref, b1_ref, w2_ref, b2_ref, o_ref):
    x = x_ref[...].reshape(BB * P, N * C)
    ms = jnp.where(m_ref[...].reshape(BB * P, N), 0.0, SENT)
    w1 = w1_ref[...].astype(jnp.bfloat16)
    w2 = w2_ref[...].astype(jnp.bfloat16)
    b1 = b1_ref[...].astype(jnp.bfloat16)
    acc = None
    for j in range(N):
        xj = x[:, j * C : (j + 1) * C].astype(jnp.bfloat16)
        d1 = jnp.dot(xj, w1, preferred_element_type=jnp.float32)
        h1 = jnp.maximum(d1.astype(jnp.bfloat16) + b1, jnp.bfloat16(0.0))
        g2 = jnp.dot(h1, w2, preferred_element_type=jnp.float32)
        cand = g2 + ms[:, j : j + 1]
        acc = cand if acc is None else jnp.maximum(acc, cand)
    o_ref[...] = jnp.where(acc < SENT / 2, 0.0, acc + b2_ref[...]).reshape(BB, P, H)


@jax.jit
def kernel(polylines, polylines_mask, W1, b1, W2, b2):
    x = polylines.reshape(B, P, N * C)
    b1r = b1.reshape(1, H)
    b2r = b2.reshape(1, H)
    out = pl.pallas_call(
        _mlp_pool_kernel,
        grid=(B // BB,),
        in_specs=[
            pl.BlockSpec((BB, P, N * C), lambda g: (g, 0, 0)),
            pl.BlockSpec((BB, P, N), lambda g: (g, 0, 0)),
            pl.BlockSpec((C, H), lambda g: (0, 0)),
            pl.BlockSpec((1, H), lambda g: (0, 0)),
            pl.BlockSpec((H, H), lambda g: (0, 0)),
            pl.BlockSpec((1, H), lambda g: (0, 0)),
        ],
        out_specs=pl.BlockSpec((BB, P, H), lambda g: (g, 0, 0)),
        out_shape=jax.ShapeDtypeStruct((B, P, H), jnp.float32),
    )(x, polylines_mask, W1, b1r, W2, b2r)
    return out


# speedup vs baseline: 1.0503x; 1.0167x over previous
import jax
import jax.numpy as jnp
from jax.experimental import pallas as pl

B, P, N, C, H = 16, 512, 20, 9, 256
SENT = -1073741824.0
BB = 4


def _mlp_pool_kernel(x_ref, m_ref, w1_ref, b1_ref, w2_ref, b2_ref, o_ref):
    x = x_ref[...].reshape(BB * P, N * C)
    ms = jnp.where(m_ref[...].reshape(BB * P, N), 0.0, SENT)
    w1 = w1_ref[...].astype(jnp.bfloat16)
    w2 = w2_ref[...].astype(jnp.bfloat16)
    b1 = b1_ref[...].astype(jnp.bfloat16)
    acc = None
    for j in range(N):
        xj = x[:, j * C : (j + 1) * C].astype(jnp.bfloat16)
        d1 = jnp.dot(xj, w1, preferred_element_type=jnp.float32)
        h1 = jnp.maximum(d1.astype(jnp.bfloat16) + b1, jnp.bfloat16(0.0))
        g2 = jnp.dot(h1, w2, preferred_element_type=jnp.float32)
        cand = g2 + ms[:, j : j + 1]
        acc = cand if acc is None else jnp.maximum(acc, cand)
    o_ref[...] = jnp.where(acc < SENT / 2, 0.0, acc + b2_ref[...]).reshape(BB, P, H)


@jax.jit
def kernel(polylines, polylines_mask, W1, b1, W2, b2):
    x = polylines.reshape(B, P, N * C)
    b1r = b1.reshape(1, H)
    b2r = b2.reshape(1, H)
    out = pl.pallas_call(
        _mlp_pool_kernel,
        grid=(B // BB,),
        in_specs=[
            pl.BlockSpec((BB, P, N * C), lambda g: (g, 0, 0)),
            pl.BlockSpec((BB, P, N), lambda g: (g, 0, 0)),
            pl.BlockSpec((C, H), lambda g: (0, 0)),
            pl.BlockSpec((1, H), lambda g: (0, 0)),
            pl.BlockSpec((H, H), lambda g: (0, 0)),
            pl.BlockSpec((1, H), lambda g: (0, 0)),
        ],
        out_specs=pl.BlockSpec((BB, P, H), lambda g: (g, 0, 0)),
        out_shape=jax.ShapeDtypeStruct((B, P, H), jnp.float32),
    )(x, polylines_mask, W1, b1r, W2, b2r)
    return out
